# baseline (speedup 1.0000x reference)
import jax
import jax.numpy as jnp
from jax.experimental import pallas as pl
from jax.experimental.pallas import tpu as pltpu

N_CHUNK = 4


def _mm_kernel(x_ref, w_hbm, out_ref, wv_ref, sems):
    b = pl.program_id(0)
    o = pl.program_id(1)
    O = w_hbm.shape[0]
    C = O // N_CHUNK

    @pl.when((b == 0) & (o == 0))
    def _start_w_copies():
        for q in range(N_CHUNK):
            pltpu.make_async_copy(
                w_hbm.at[pl.ds(q * C, C), :], wv_ref.at[q],
                sems.at[q]).start()

    @pl.when(b == 0)
    def _wait_w_chunk():
        for q in range(N_CHUNK):
            @pl.when(o == q)
            def _():
                pltpu.make_async_copy(
                    w_hbm.at[pl.ds(q * C, C), :], wv_ref.at[q],
                    sems.at[q]).wait()

    out_ref[0] = jax.lax.dot_general(
        wv_ref[o], x_ref[0],
        (((1,), (1,)), ((), ())),
        preferred_element_type=jnp.float32,
    )


@jax.jit
def kernel(x, weight):
    B, S, I = x.shape
    O = weight.shape[0]
    O_BLK = O // N_CHUNK
    return pl.pallas_call(
        _mm_kernel,
        grid=(B, N_CHUNK),
        in_specs=[
            pl.BlockSpec((1, S, I), lambda b, o: (b, 0, 0)),
            pl.BlockSpec(memory_space=pl.ANY),
        ],
        out_specs=pl.BlockSpec((1, O_BLK, S), lambda b, o: (b, o, 0)),
        out_shape=jax.ShapeDtypeStruct((B, O, S), jnp.float32),
        scratch_shapes=[
            pltpu.VMEM((N_CHUNK, O_BLK, I), jnp.float32),
            pltpu.SemaphoreType.DMA((N_CHUNK,)),
        ],
        compiler_params=pltpu.CompilerParams(
            dimension_semantics=("arbitrary", "arbitrary"),
        ),
    )(x, weight)


# speedup vs baseline: 1.1894x; 1.1894x over previous
import jax
import jax.numpy as jnp
from jax.experimental import pallas as pl
from jax.experimental.pallas import tpu as pltpu

N_CHUNK = 4


def _mm_kernel(x_ref, w_hbm, out_ref, wv_ref, sems):
    first = (pl.program_id(0) == 0) & (pl.program_id(1) == 0)
    O = wv_ref.shape[0]
    C = O // N_CHUNK

    @pl.when(first)
    def _first_step():
        for q in range(N_CHUNK):
            pltpu.make_async_copy(
                w_hbm.at[pl.ds(q * C, C), :], wv_ref.at[pl.ds(q * C, C), :],
                sems.at[q]).start()
        for q in range(N_CHUNK):
            pltpu.make_async_copy(
                w_hbm.at[pl.ds(q * C, C), :], wv_ref.at[pl.ds(q * C, C), :],
                sems.at[q]).wait()
            out_ref[0, pl.ds(q * C, C), :] = jax.lax.dot_general(
                wv_ref[pl.ds(q * C, C), :], x_ref[0],
                (((1,), (1,)), ((), ())), preferred_element_type=jnp.float32)

    @pl.when(jnp.logical_not(first))
    def _steady_state():
        out_ref[0] = jax.lax.dot_general(
            wv_ref[...], x_ref[0],
            (((1,), (1,)), ((), ())), preferred_element_type=jnp.float32)


@jax.jit
def kernel(x, weight):
    B, S, I = x.shape
    O = weight.shape[0]
    S_BLK = min(S, 512)

    grid = (B, S // S_BLK)
    return pl.pallas_call(
        _mm_kernel,
        grid=grid,
        in_specs=[
            pl.BlockSpec((1, S_BLK, I), lambda b, s: (b, s, 0)),
            pl.BlockSpec(memory_space=pl.ANY),
        ],
        out_specs=pl.BlockSpec((1, O, S_BLK), lambda b, s: (b, 0, s)),
        out_shape=jax.ShapeDtypeStruct((B, O, S), jnp.float32),
        scratch_shapes=[
            pltpu.VMEM((O, I), jnp.float32),
            pltpu.SemaphoreType.DMA((N_CHUNK,)),
        ],
        compiler_params=pltpu.CompilerParams(
            dimension_semantics=("arbitrary", "arbitrary"),
        ),
    )(x, weight)
